# Initial kernel scaffold; baseline (speedup 1.0000x reference)
import jax
import jax.numpy as jnp
from jax.experimental import pallas as pl


def kernel(srcPC, dstPC):
    raise NotImplementedError("write your pallas kernel here")


# speedup vs baseline: 338.9460x; 338.9460x over previous
import functools

import jax
import jax.numpy as jnp
from jax.experimental import pallas as pl
from jax.experimental.pallas import tpu as pltpu

N_PATCH = 8
K_NN = 16
NEWTON_ITERS = 18
INTERPRET = False


def _cloud_props(co, ct, k):
    P = co.shape[1]
    x = co[0:1, :]
    y = co[1:2, :]
    z = co[2:3, :]
    xc = ct[:, 0:1]
    yc = ct[:, 1:2]
    zc = ct[:, 2:3]

    dx = xc - x
    dy = yc - y
    dz = zc - z
    d2 = dx * dx + dy * dy + dz * dz

    dw = d2
    t = None
    for it in range(k):
        t = jnp.min(dw, axis=0, keepdims=True)
        if it < k - 1:
            dw = jnp.where(dw == t, jnp.float32(jnp.inf), dw)

    maskf = jnp.where(d2 <= t, jnp.float32(1.0), jnp.float32(0.0))

    one = jnp.ones_like(x)
    feats = jnp.concatenate(
        [x, y, z, x * x, y * y, z * z, x * y, x * z, y * z, one], axis=0
    )
    S = jnp.dot(feats, maskf, preferred_element_type=jnp.float32)

    Sx = S[0:1]
    Sy = S[1:2]
    Sz = S[2:3]
    cn = S[9:10]
    cxx = S[3:4] - 2.0 * x * Sx + cn * x * x
    cyy = S[4:5] - 2.0 * y * Sy + cn * y * y
    czz = S[5:6] - 2.0 * z * Sz + cn * z * z
    cxy = S[6:7] - x * Sy - y * Sx + cn * x * y
    cxz = S[7:8] - x * Sz - z * Sx + cn * x * z
    cyz = S[8:9] - y * Sz - z * Sy + cn * y * z

    c2 = cxx + cyy + czz
    c1 = (cxx * cyy - cxy * cxy) + (cxx * czz - cxz * cxz) + (cyy * czz - cyz * cyz)
    c0 = (
        cxx * (cyy * czz - cyz * cyz)
        - cxy * (cxy * czz - cyz * cxz)
        + cxz * (cxy * cyz - cyy * cxz)
    )
    lam = -0.01 * c2 - jnp.float32(1e-12)
    for _ in range(NEWTON_ITERS):
        fv = ((c2 - lam) * lam - c1) * lam + c0
        fp = (2.0 * c2 - 3.0 * lam) * lam - c1
        fp = jnp.minimum(fp, jnp.float32(-1e-30))
        lam = lam - fv / fp

    m00 = cxx - lam
    m11 = cyy - lam
    m22 = czz - lam
    a01x = cxy * cyz - cxz * m11
    a01y = cxz * cxy - m00 * cyz
    a01z = m00 * m11 - cxy * cxy
    a02x = cxy * m22 - cxz * cyz
    a02y = cxz * cxz - m00 * m22
    a02z = m00 * cyz - cxy * cxz
    a12x = m11 * m22 - cyz * cyz
    a12y = cyz * cxz - cxy * m22
    a12z = cxy * cyz - m11 * cxz
    n01 = a01x * a01x + a01y * a01y + a01z * a01z
    n02 = a02x * a02x + a02y * a02y + a02z * a02z
    n12 = a12x * a12x + a12y * a12y + a12z * a12z

    use02 = n02 > n01
    vx = jnp.where(use02, a02x, a01x)
    vy = jnp.where(use02, a02y, a01y)
    vz = jnp.where(use02, a02z, a01z)
    nb = jnp.maximum(n01, n02)
    use12 = n12 > nb
    vx = jnp.where(use12, a12x, vx)
    vy = jnp.where(use12, a12y, vy)
    vz = jnp.where(use12, a12z, vz)
    nb = jnp.maximum(nb, n12)

    inv = jax.lax.rsqrt(nb + jnp.float32(1e-38))
    anx = jnp.abs(vx) * inv
    any_ = jnp.abs(vy) * inv
    anz = jnp.abs(vz) * inv
    sv = lam / jnp.maximum(c2, jnp.float32(1e-38))
    return anx, any_, anz, sv


def _pair_body(k, coords_ref, coords_t_ref, out_ref):
    step = pl.program_id(0)
    sx, sy, sz, ssv = _cloud_props(coords_ref[0, 0], coords_t_ref[0, 0], k)
    dx_, dy_, dz_, dsv = _cloud_props(coords_ref[0, 1], coords_t_ref[0, 1], k)

    nl = jnp.sqrt((sx - dx_) ** 2 + (sy - dy_) ** 2 + (sz - dz_) ** 2)
    svl = jnp.abs(ssv - dsv)
    zeros = jnp.zeros_like(nl)
    acc = jnp.concatenate([nl, svl, zeros, zeros, zeros, zeros, zeros, zeros], axis=0)

    @pl.when(step == 0)
    def _():
        out_ref[...] = jnp.zeros_like(out_ref)

    out_ref[...] += acc


def kernel(srcPC, dstPC):
    B, N, _ = srcPC.shape
    n_pairs = B * N_PATCH
    P = N // N_PATCH

    s = srcPC.reshape(n_pairs, P, 3)
    d = dstPC.reshape(n_pairs, P, 3)
    pts = jnp.stack([s, d], axis=1)
    pts = jnp.pad(pts, ((0, 0), (0, 0), (0, 0), (0, 5)))
    coords_t = pts
    coords = jnp.swapaxes(pts, 2, 3)

    body = functools.partial(_pair_body, K_NN)
    res = pl.pallas_call(
        body,
        grid=(n_pairs,),
        in_specs=[
            pl.BlockSpec((1, 2, 8, P), lambda i: (i, 0, 0, 0)),
            pl.BlockSpec((1, 2, P, 8), lambda i: (i, 0, 0, 0)),
        ],
        out_specs=pl.BlockSpec((8, P), lambda i: (0, 0)),
        out_shape=jax.ShapeDtypeStruct((8, P), jnp.float32),
        compiler_params=pltpu.CompilerParams(
            dimension_semantics=("arbitrary",),
        ),
        interpret=INTERPRET,
    )(coords, coords_t)

    npts = jnp.float32(B * N)
    normal_loss = jnp.sum(res[0]) / npts * jnp.float32(1.0)
    surf_loss = jnp.sum(res[1]) / npts * jnp.float32(1.0)
    return jnp.stack([normal_loss, surf_loss])
